# Initial kernel scaffold; baseline (speedup 1.0000x reference)
import jax
import jax.numpy as jnp
from jax.experimental import pallas as pl


def kernel(inputs, pos_table):
    raise NotImplementedError("write your pallas kernel here")


# speedup vs baseline: 1.4398x; 1.4398x over previous
import jax
import jax.numpy as jnp
from jax.experimental import pallas as pl

_BLOCK_S = 512


def _add_kernel(x_ref, t_ref, o_ref):
    o_ref[...] = x_ref[...] + t_ref[...]


def kernel(inputs, pos_table):
    batch, seq_len, dim = inputs.shape
    grid = (seq_len // _BLOCK_S, batch)
    return pl.pallas_call(
        _add_kernel,
        grid=grid,
        in_specs=[
            pl.BlockSpec((1, _BLOCK_S, dim), lambda s, b: (b, s, 0)),
            pl.BlockSpec((_BLOCK_S, dim), lambda s, b: (s, 0)),
        ],
        out_specs=pl.BlockSpec((1, _BLOCK_S, dim), lambda s, b: (b, s, 0)),
        out_shape=jax.ShapeDtypeStruct(inputs.shape, inputs.dtype),
    )(inputs, pos_table)


# speedup vs baseline: 1.6782x; 1.1656x over previous
import jax
import jax.numpy as jnp
from jax.experimental import pallas as pl

_BLOCK_S = 1024


def _add_kernel(x_ref, t_ref, o_ref):
    o_ref[...] = x_ref[...] + t_ref[...]


def kernel(inputs, pos_table):
    batch, seq_len, dim = inputs.shape
    grid = (seq_len // _BLOCK_S, batch)
    return pl.pallas_call(
        _add_kernel,
        grid=grid,
        in_specs=[
            pl.BlockSpec((1, _BLOCK_S, dim), lambda s, b: (b, s, 0)),
            pl.BlockSpec((_BLOCK_S, dim), lambda s, b: (s, 0)),
        ],
        out_specs=pl.BlockSpec((1, _BLOCK_S, dim), lambda s, b: (b, s, 0)),
        out_shape=jax.ShapeDtypeStruct(inputs.shape, inputs.dtype),
    )(inputs, pos_table)


# speedup vs baseline: 1.7968x; 1.0707x over previous
import jax
import jax.numpy as jnp
from jax.experimental import pallas as pl

_BLOCK_S = 2048


def _add_kernel(x_ref, t_ref, o_ref):
    o_ref[...] = x_ref[...] + t_ref[...]


def kernel(inputs, pos_table):
    batch, seq_len, dim = inputs.shape
    grid = (seq_len // _BLOCK_S, batch)
    return pl.pallas_call(
        _add_kernel,
        grid=grid,
        in_specs=[
            pl.BlockSpec((1, _BLOCK_S, dim), lambda s, b: (b, s, 0)),
            pl.BlockSpec((_BLOCK_S, dim), lambda s, b: (s, 0)),
        ],
        out_specs=pl.BlockSpec((1, _BLOCK_S, dim), lambda s, b: (b, s, 0)),
        out_shape=jax.ShapeDtypeStruct(inputs.shape, inputs.dtype),
    )(inputs, pos_table)


# speedup vs baseline: 1.7998x; 1.0016x over previous
import jax
import jax.numpy as jnp
from jax.experimental import pallas as pl

_BLOCK_S = 1024


def _add_kernel(x_ref, t_ref, o_ref):
    o_ref[...] = x_ref[...] + t_ref[...][None]


def kernel(inputs, pos_table):
    batch, seq_len, dim = inputs.shape
    grid = (seq_len // _BLOCK_S,)
    return pl.pallas_call(
        _add_kernel,
        grid=grid,
        in_specs=[
            pl.BlockSpec((batch, _BLOCK_S, dim), lambda s: (0, s, 0)),
            pl.BlockSpec((_BLOCK_S, dim), lambda s: (s, 0)),
        ],
        out_specs=pl.BlockSpec((batch, _BLOCK_S, dim), lambda s: (0, s, 0)),
        out_shape=jax.ShapeDtypeStruct(inputs.shape, inputs.dtype),
    )(inputs, pos_table)
